# Initial kernel scaffold; baseline (speedup 1.0000x reference)
import functools

import jax
import jax.numpy as jnp
from jax import lax
from jax.experimental import pallas as pl
from jax.experimental.pallas import tpu as pltpu
from jax.experimental.pallas import tpu_sc as plsc

_N_USERS = 50000
_N_ITEMS = 50000
_DIM = 32
_N_NODES = _N_USERS + _N_ITEMS
_N_EDGES = 1600000

_NC = 2
_NS = 16
_CH = 128
_CPT = -(-_N_EDGES // (_CH * _NS))
_NCH = _CPT * _NS
_WORDS = 3 * _CH

_ROWS_PER_CORE = _N_NODES // _NC
_ACC_ROWS = 51200
_DUMP_ROW = _ROWS_PER_CORE
_ZROWS = _ACC_ROWS // _NS
_DRAIN = _ROWS_PER_CORE // _NS


def _scale_and_index(idxb, gath, lidx, base_row):
  def g_body(g, carry):
    off = g * 16
    rows16 = idxb[pl.ds(off, 16)]
    local = rows16 - base_row
    ok = (local >= 0) & (local < _ROWS_PER_CORE)
    lidx[pl.ds(off, 16)] = jnp.where(ok, local, _DUMP_ROW)
    for i in range(16):
      e = off + i
      m = plsc.load_gather(idxb, [jnp.full((16,), 2 * _CH, jnp.int32) + e])
      m = plsc.bitcast(m, jnp.float32)
      gath[e, pl.ds(0, 16)] = gath[e, pl.ds(0, 16)] * m
      gath[e, pl.ds(16, 16)] = gath[e, pl.ds(16, 16)] * m
    return carry

  lax.fori_loop(0, _CH // 16, g_body, 0)


def _layer_body(packed_h, zeros_h, emb_h, out_h,
                accum, idxb0, idxb1, lidx0, lidx1, g0, g1, sem0, sem1):
  cid = lax.axis_index("c")
  sid = lax.axis_index("s")
  base_row = cid * _ROWS_PER_CORE

  pltpu.sync_copy(zeros_h, accum.at[pl.ds(sid * _ZROWS, _ZROWS)])
  plsc.subcore_barrier()

  c0 = sid * _CPT

  def load_idx(chunk, idxb):
    pltpu.sync_copy(packed_h.at[pl.ds(chunk * _WORDS, _WORDS)], idxb)

  def start_gather(idxb, gath, sem):
    pltpu.async_copy(emb_h.at[idxb.at[pl.ds(_CH, _CH)]], gath, sem)

  def wait_gather(idxb, gath, sem):
    pltpu.make_async_copy(emb_h.at[idxb.at[pl.ds(_CH, _CH)]], gath, sem).wait()

  load_idx(c0, idxb0)
  start_gather(idxb0, g0, sem0)
  load_idx(c0 + 1, idxb1)
  start_gather(idxb1, g1, sem1)

  def slot(chunk_next, idxb, lidx, gath, sem, issue_next):
    wait_gather(idxb, gath, sem)
    _scale_and_index(idxb, gath, lidx, base_row)
    pltpu.sync_copy(gath, accum.at[lidx], add=True)
    if issue_next:
      load_idx(chunk_next, idxb)
      start_gather(idxb, gath, sem)

  def loop_body(j, carry):
    ca = c0 + 2 * j
    slot(ca + 2, idxb0, lidx0, g0, sem0, True)
    slot(ca + 3, idxb1, lidx1, g1, sem1, True)
    return carry

  lax.fori_loop(0, _CPT // 2 - 1, loop_body, 0)
  slot(0, idxb0, lidx0, g0, sem0, False)
  slot(0, idxb1, lidx1, g1, sem1, False)

  plsc.subcore_barrier()
  pltpu.sync_copy(
      accum.at[pl.ds(sid * _DRAIN, _DRAIN)],
      out_h.at[pl.ds(cid * _ROWS_PER_CORE + sid * _DRAIN, _DRAIN)])


_sc_layer = functools.partial(
    pl.kernel,
    out_type=jax.ShapeDtypeStruct((_N_NODES, _DIM), jnp.float32),
    mesh=plsc.VectorSubcoreMesh(
        core_axis_name="c", subcore_axis_name="s",
        num_cores=_NC, num_subcores=_NS),
    scratch_types=[
        pltpu.VMEM_SHARED((_ACC_ROWS, _DIM), jnp.float32),
        pltpu.VMEM((_WORDS,), jnp.int32),
        pltpu.VMEM((_WORDS,), jnp.int32),
        pltpu.VMEM((_CH,), jnp.int32),
        pltpu.VMEM((_CH,), jnp.int32),
        pltpu.VMEM((_CH, _DIM), jnp.float32),
        pltpu.VMEM((_CH, _DIM), jnp.float32),
        pltpu.SemaphoreType.DMA,
        pltpu.SemaphoreType.DMA,
    ],
)(_layer_body)


def _mean_body(a, b, c, d, o):
  o[...] = (a[...] + b[...] + c[...] + d[...]) * 0.25


_mean4 = pl.pallas_call(
    _mean_body,
    grid=(50,),
    in_specs=[pl.BlockSpec((_N_NODES // 50, _DIM), lambda i: (i, 0))] * 4,
    out_specs=pl.BlockSpec((_N_NODES // 50, _DIM), lambda i: (i, 0)),
    out_shape=jax.ShapeDtypeStruct((_N_NODES, _DIM), jnp.float32),
)


def _pack_edges(adj_indices, adj_values):
  pad = _NCH * _CH - _N_EDGES
  rows = jnp.concatenate([adj_indices[0], jnp.zeros((pad,), jnp.int32)])
  cols = jnp.concatenate([adj_indices[1], jnp.zeros((pad,), jnp.int32)])
  vals = jnp.concatenate([adj_values, jnp.zeros((pad,), jnp.float32)])
  vbits = lax.bitcast_convert_type(vals, jnp.int32)
  packed = jnp.stack(
      [rows.reshape(_NCH, _CH), cols.reshape(_NCH, _CH),
       vbits.reshape(_NCH, _CH)], axis=1)
  return packed.reshape(-1)


def kernel(adj_indices, adj_values, user_emb, item_emb):
  packed = _pack_edges(adj_indices, adj_values)
  zeros = jnp.zeros((_ZROWS, _DIM), jnp.float32)
  emb0 = jnp.concatenate([user_emb, item_emb], axis=0)
  emb1 = _sc_layer(packed, zeros, emb0)
  emb2 = _sc_layer(packed, zeros, emb1)
  emb3 = _sc_layer(packed, zeros, emb2)
  out = _mean4(emb0, emb1, emb2, emb3)
  return (out[:_N_USERS], out[_N_USERS:])


# speedup vs baseline: 6.4980x; 6.4980x over previous
import functools

import jax
import jax.numpy as jnp
from jax import lax
from jax.experimental import pallas as pl
from jax.experimental.pallas import tpu as pltpu
from jax.experimental.pallas import tpu_sc as plsc

_N_USERS = 50000
_N_ITEMS = 50000
_DIM = 32
_N_NODES = _N_USERS + _N_ITEMS
_N_EDGES = 1600000

_NC = 2
_NS = 16
_CH = 128
_CPT = -(-_N_EDGES // (_CH * _NS))
_NCH = _CPT * _NS
_WORDS = 3 * _CH

_ROWS_PER_CORE = _N_NODES // _NC
_ACC_ROWS = 51200
_DUMP_ROW = _ROWS_PER_CORE
_ZROWS = _ACC_ROWS // _NS
_DRAIN = 3128
_DRAIN_LAST = _ROWS_PER_CORE - 15 * _DRAIN


def _scale_and_index(idxb, gath, lidx, base_row):
  def g_body(g, carry):
    off = g * 16
    rows16 = idxb[pl.ds(off, 16)]
    local = rows16 - base_row
    ok = (local >= 0) & (local < _ROWS_PER_CORE)
    lidx[pl.ds(off, 16)] = jnp.where(ok, local, _DUMP_ROW)
    for i in range(16):
      e = off + i
      m = plsc.load_gather(idxb, [jnp.full((16,), 2 * _CH, jnp.int32) + e])
      m = plsc.bitcast(m, jnp.float32)
      gath[e, pl.ds(0, 16)] = gath[e, pl.ds(0, 16)] * m
      gath[e, pl.ds(16, 16)] = gath[e, pl.ds(16, 16)] * m
    return carry

  lax.fori_loop(0, _CH // 16, g_body, 0)


def _layer_body(packed_h, zeros_h, emb_h, out_h,
                accum, idxb0, idxb1, lidx0, lidx1, g0, g1, sem0, sem1):
  cid = lax.axis_index("c")
  sid = lax.axis_index("s")
  base_row = cid * _ROWS_PER_CORE

  pltpu.sync_copy(zeros_h, accum.at[pl.ds(sid * _ZROWS, _ZROWS)])
  plsc.subcore_barrier()

  c0 = sid * _CPT

  def load_idx(chunk, idxb):
    pltpu.sync_copy(packed_h.at[pl.ds(chunk * _WORDS, _WORDS)], idxb)

  def start_gather(idxb, gath, sem):
    pltpu.async_copy(emb_h.at[idxb.at[pl.ds(_CH, _CH)]], gath, sem)

  def wait_gather(idxb, gath, sem):
    pltpu.make_async_copy(emb_h.at[idxb.at[pl.ds(_CH, _CH)]], gath, sem).wait()

  load_idx(c0, idxb0)
  start_gather(idxb0, g0, sem0)
  load_idx(c0 + 1, idxb1)
  start_gather(idxb1, g1, sem1)

  def slot(chunk_next, idxb, lidx, gath, sem, issue_next):
    wait_gather(idxb, gath, sem)
    _scale_and_index(idxb, gath, lidx, base_row)
    pltpu.sync_copy(gath, accum.at[lidx], add=True)
    if issue_next:
      load_idx(chunk_next, idxb)
      start_gather(idxb, gath, sem)

  def loop_body(j, carry):
    ca = c0 + 2 * j
    slot(ca + 2, idxb0, lidx0, g0, sem0, True)
    slot(ca + 3, idxb1, lidx1, g1, sem1, True)
    return carry

  lax.fori_loop(0, _CPT // 2 - 1, loop_body, 0)
  slot(0, idxb0, lidx0, g0, sem0, False)
  slot(0, idxb1, lidx1, g1, sem1, False)

  plsc.subcore_barrier()

  @pl.when(sid < _NS - 1)
  def _drain_main():
    pltpu.sync_copy(
        accum.at[pl.ds(sid * _DRAIN, _DRAIN)],
        out_h.at[pl.ds(cid * _ROWS_PER_CORE + sid * _DRAIN, _DRAIN)])

  @pl.when(sid == _NS - 1)
  def _drain_last():
    pltpu.sync_copy(
        accum.at[pl.ds((_NS - 1) * _DRAIN, _DRAIN_LAST)],
        out_h.at[pl.ds(cid * _ROWS_PER_CORE + (_NS - 1) * _DRAIN,
                       _DRAIN_LAST)])


_sc_layer = functools.partial(
    pl.kernel,
    out_type=jax.ShapeDtypeStruct((_N_NODES, _DIM), jnp.float32),
    mesh=plsc.VectorSubcoreMesh(
        core_axis_name="c", subcore_axis_name="s",
        num_cores=_NC, num_subcores=_NS),
    scratch_types=[
        pltpu.VMEM_SHARED((_ACC_ROWS, _DIM), jnp.float32),
        pltpu.VMEM((_WORDS,), jnp.int32),
        pltpu.VMEM((_WORDS,), jnp.int32),
        pltpu.VMEM((_CH,), jnp.int32),
        pltpu.VMEM((_CH,), jnp.int32),
        pltpu.VMEM((_CH, _DIM), jnp.float32),
        pltpu.VMEM((_CH, _DIM), jnp.float32),
        pltpu.SemaphoreType.DMA,
        pltpu.SemaphoreType.DMA,
    ],
    compiler_params=pltpu.CompilerParams(
        needs_layout_passes=False, use_tc_tiling_on_sc=False),
)(_layer_body)


def _mean_body(a, b, c, d, o):
  o[...] = (a[...] + b[...] + c[...] + d[...]) * 0.25


_mean4 = pl.pallas_call(
    _mean_body,
    grid=(50,),
    in_specs=[pl.BlockSpec((_N_NODES // 50, _DIM), lambda i: (i, 0))] * 4,
    out_specs=pl.BlockSpec((_N_NODES // 50, _DIM), lambda i: (i, 0)),
    out_shape=jax.ShapeDtypeStruct((_N_NODES, _DIM), jnp.float32),
)


def _pack_edges(adj_indices, adj_values):
  pad = _NCH * _CH - _N_EDGES
  rows = jnp.concatenate([adj_indices[0], jnp.zeros((pad,), jnp.int32)])
  cols = jnp.concatenate([adj_indices[1], jnp.zeros((pad,), jnp.int32)])
  vals = jnp.concatenate([adj_values, jnp.zeros((pad,), jnp.float32)])
  vbits = lax.bitcast_convert_type(vals, jnp.int32)
  packed = jnp.stack(
      [rows.reshape(_NCH, _CH), cols.reshape(_NCH, _CH),
       vbits.reshape(_NCH, _CH)], axis=1)
  return packed.reshape(-1)


def kernel(adj_indices, adj_values, user_emb, item_emb):
  packed = _pack_edges(adj_indices, adj_values)
  zeros = jnp.zeros((_ZROWS, _DIM), jnp.float32)
  emb0 = jnp.concatenate([user_emb, item_emb], axis=0)
  emb1 = _sc_layer(packed, zeros, emb0)
  emb2 = _sc_layer(packed, zeros, emb1)
  emb3 = _sc_layer(packed, zeros, emb2)
  out = _mean4(emb0, emb1, emb2, emb3)
  return (out[:_N_USERS], out[_N_USERS:])
